# baseline (speedup 1.0000x reference)
import functools

import jax
import jax.numpy as jnp
from jax import lax
from jax.experimental import pallas as pl
from jax.experimental.pallas import tpu as pltpu
from jax.experimental.pallas import tpu_sc as plsc

L = 16
NC, NS = 2, 16
NW = NC * NS
BATCH = 16384
BPW = BATCH // NW
D = 32
CHUNK = 128
NCHUNK = BPW // CHUNK
BLK = 65536


def _score_body(t_ref, w_ref, o_ref):
    x3 = t_ref[...].reshape(D, BLK // 128, 128)
    acc = x3[0] * w_ref[0, 0]
    for d in range(1, D):
        acc = acc + x3[d] * w_ref[d, 0]
    o_ref[...] = acc


def _scores(tab_t, w_col, nblk):
    return pl.pallas_call(
        _score_body,
        grid=(nblk,),
        in_specs=[
            pl.BlockSpec((D, BLK), lambda i: (0, i)),
            pl.BlockSpec((D, 1), lambda i: (0, 0)),
        ],
        out_specs=pl.BlockSpec((BLK // 128, 128), lambda i: (i, 0)),
        out_shape=jax.ShapeDtypeStruct((nblk * (BLK // 128), 128), jnp.float32),
    )(tab_t, w_col)


_mesh = plsc.VectorSubcoreMesh(core_axis_name="c", subcore_axis_name="s")


@functools.partial(
    pl.kernel,
    out_type=jax.ShapeDtypeStruct((NW, BPW), jnp.float32),
    mesh=_mesh,
    scratch_types=[
        pltpu.VMEM((NCHUNK, CHUNK), jnp.int32),
        pltpu.VMEM((NCHUNK, CHUNK), jnp.int32),
        pltpu.VMEM((NCHUNK, CHUNK), jnp.float32),
        pltpu.VMEM((NCHUNK, CHUNK), jnp.float32),
        pltpu.VMEM((L,), jnp.float32),
        pltpu.VMEM((BPW,), jnp.float32),
        pltpu.SemaphoreType.DMA,
    ],
    compiler_params=pltpu.CompilerParams(needs_layout_passes=False),
)
def _sc_pick(users_hbm, movies_hbm, su_hbm, sm_hbm, b_hbm, out_hbm,
             idx_u, idx_m, gu, gm, b_v, out_v, sem):
    wid = lax.axis_index("s") * NC + lax.axis_index("c")

    pltpu.sync_copy(users_hbm.at[wid], idx_u)
    pltpu.sync_copy(movies_hbm.at[wid], idx_m)
    pltpu.sync_copy(b_hbm, b_v)

    copies = []
    for j in range(NCHUNK):
        copies.append(pltpu.async_copy(su_hbm.at[idx_u.at[j]], gu.at[j], sem))
        copies.append(pltpu.async_copy(sm_hbm.at[idx_m.at[j]], gm.at[j], sem))
    for c in copies:
        c.wait()

    bvec = b_v[...]
    for j in range(NCHUNK):
        for k in range(CHUNK // L):
            s = pl.ds(k * L, L)
            out_v[pl.ds(j * CHUNK + k * L, L)] = gu[j, s] + gm[j, s] + bvec

    pltpu.sync_copy(out_v, out_hbm.at[wid])


def kernel(users, movies, user_table, movie_table, W, b):
    n_u = user_table.shape[0]
    n_m = movie_table.shape[0]
    nblk_u = -(-n_u // BLK)
    nblk_m = -(-n_m // BLK)

    su = _scores(user_table.T, W[:D], nblk_u).reshape(-1)
    sm = _scores(movie_table.T, W[D:], nblk_m).reshape(-1)

    users_r = users.astype(jnp.int32).reshape(NW, NCHUNK, CHUNK)
    movies_r = movies.astype(jnp.int32).reshape(NW, NCHUNK, CHUNK)
    b16 = jnp.broadcast_to(b.reshape(1), (L,))
    out = _sc_pick(users_r, movies_r, su, sm, b16)
    return out.reshape(BATCH, 1)


# speedup vs baseline: 1.0912x; 1.0912x over previous
import functools

import jax
import jax.numpy as jnp
from jax import lax
from jax.experimental import pallas as pl
from jax.experimental.pallas import tpu as pltpu
from jax.experimental.pallas import tpu_sc as plsc

L = 16
NC, NS = 2, 16
NW = NC * NS
BATCH = 16384
BPW = BATCH // NW
D = 32
CHUNK = 128
NCHUNK = BPW // CHUNK
BLK = 65536


def _score_body(ta_ref, tb_ref, w_ref, o_ref):
    w = w_ref[...]
    xa = ta_ref[...].reshape(D, BLK // 128, 128)
    xb = tb_ref[...].reshape(D, BLK // 128, 128)
    o_ref[: BLK // 128] = jnp.sum(xa * w[:, :, None], axis=0)
    o_ref[BLK // 128 :] = jnp.sum(xb * w[:, :, None], axis=0)


def _scores(tab_t, w_col, nblk):
    npair = -(-nblk // 2)
    return pl.pallas_call(
        _score_body,
        grid=(npair,),
        in_specs=[
            pl.BlockSpec((D, BLK), lambda i: (0, 2 * i)),
            pl.BlockSpec((D, BLK), lambda i: (0, 2 * i + 1)),
            pl.BlockSpec((D, 1), lambda i: (0, 0)),
        ],
        out_specs=pl.BlockSpec((2 * (BLK // 128), 128), lambda i: (i, 0)),
        out_shape=jax.ShapeDtypeStruct(
            (npair * 2 * (BLK // 128), 128), jnp.float32),
    )(tab_t, tab_t, w_col)


_mesh = plsc.VectorSubcoreMesh(core_axis_name="c", subcore_axis_name="s")


@functools.partial(
    pl.kernel,
    out_type=jax.ShapeDtypeStruct((NW, BPW), jnp.float32),
    mesh=_mesh,
    scratch_types=[
        pltpu.VMEM((NCHUNK, CHUNK), jnp.int32),
        pltpu.VMEM((NCHUNK, CHUNK), jnp.int32),
        pltpu.VMEM((NCHUNK, CHUNK), jnp.float32),
        pltpu.VMEM((NCHUNK, CHUNK), jnp.float32),
        pltpu.VMEM((L,), jnp.float32),
        pltpu.VMEM((BPW,), jnp.float32),
        pltpu.SemaphoreType.DMA,
    ],
    compiler_params=pltpu.CompilerParams(needs_layout_passes=False),
)
def _sc_pick(users_hbm, movies_hbm, su_hbm, sm_hbm, b_hbm, out_hbm,
             idx_u, idx_m, gu, gm, b_v, out_v, sem):
    wid = lax.axis_index("s") * NC + lax.axis_index("c")

    pltpu.sync_copy(users_hbm.at[wid], idx_u)
    pltpu.sync_copy(movies_hbm.at[wid], idx_m)
    pltpu.sync_copy(b_hbm, b_v)

    copies = []
    for j in range(NCHUNK):
        copies.append(pltpu.async_copy(su_hbm.at[idx_u.at[j]], gu.at[j], sem))
        copies.append(pltpu.async_copy(sm_hbm.at[idx_m.at[j]], gm.at[j], sem))
    for c in copies:
        c.wait()

    bvec = b_v[...]
    for j in range(NCHUNK):
        for k in range(CHUNK // L):
            s = pl.ds(k * L, L)
            out_v[pl.ds(j * CHUNK + k * L, L)] = gu[j, s] + gm[j, s] + bvec

    pltpu.sync_copy(out_v, out_hbm.at[wid])


def kernel(users, movies, user_table, movie_table, W, b):
    n_u = user_table.shape[0]
    n_m = movie_table.shape[0]
    nblk_u = -(-n_u // BLK)
    nblk_m = -(-n_m // BLK)

    su = _scores(user_table.T, W[:D], nblk_u).reshape(-1)
    sm = _scores(movie_table.T, W[D:], nblk_m).reshape(-1)

    users_r = users.astype(jnp.int32).reshape(NW, NCHUNK, CHUNK)
    movies_r = movies.astype(jnp.int32).reshape(NW, NCHUNK, CHUNK)
    b16 = jnp.broadcast_to(b.reshape(1), (L,))
    out = _sc_pick(users_r, movies_r, su, sm, b16)
    return out.reshape(BATCH, 1)
